# baseline (speedup 1.0000x reference)
import functools
import math

import jax
import jax.numpy as jnp
from jax import lax
from jax.experimental import pallas as pl
from jax.experimental.pallas import tpu as pltpu
from jax.experimental.pallas import tpu_sc as plsc

VOCAB = 1000000
D = 64
B = 4096
L = 200
SCALE = math.sqrt(D)
DP = 64

NUM_CORES = 2
NUM_SUBCORES = 16
NW = NUM_CORES * NUM_SUBCORES
BP = 128
LT = L // 8
NBUF = 4

CW = 256
FULLC = VOCAB // CW
AIT = FULLC // NW + 2

_mesh = plsc.VectorSubcoreMesh(core_axis_name="c", subcore_axis_name="s")


@functools.partial(
    pl.kernel,
    mesh=_mesh,
    compiler_params=pltpu.CompilerParams(needs_layout_passes=False),
    out_type=jax.ShapeDtypeStruct((VOCAB * DP,), jnp.float32),
    scratch_types=[
        [pltpu.VMEM((D, CW), jnp.float32) for _ in range(2)],
        [pltpu.VMEM((CW * DP,), jnp.float32) for _ in range(2)],
        [pltpu.SemaphoreType.DMA for _ in range(2)],
        [pltpu.SemaphoreType.DMA for _ in range(2)],
    ],
)
def _transpose_table(tt_hbm, tail_hbm, t8_hbm, inb, outb, rsem, wsem):
    wid = lax.axis_index("s") * NUM_CORES + lax.axis_index("c")

    iota = lax.broadcasted_iota(jnp.int32, (16,), 0)
    mbase = [(iota + (16 * m)) * DP for m in range(CW // 16)]

    def cid(i):
        return i * NW + wid

    def read_start(b, i):
        tc = cid(i)

        @pl.when(tc < FULLC)
        def _():
            pltpu.make_async_copy(
                tt_hbm.at[:, pl.ds(tc * CW, CW)], inb[b], rsem[b]).start()

    def read_wait(b, i):
        @pl.when(cid(i) < FULLC)
        def _():
            pltpu.make_async_copy(
                tt_hbm.at[:, pl.ds(0, CW)], inb[b], rsem[b]).wait()

    def transpose(b):
        src, dst = inb[b], outb[b]

        def dbody(d, carry):
            vals = [src[d, pl.ds(16 * m, 16)] * SCALE
                    for m in range(CW // 16)]
            idxs = [mbase[m] + d for m in range(CW // 16)]
            for m in range(CW // 16):
                plsc.store_scatter(dst, [idxs[m]], vals[m])
            return carry

        lax.fori_loop(0, D, dbody, 0, unroll=2)

    def write_start(b, i):
        tc = cid(i)

        @pl.when(tc < FULLC)
        def _():
            pltpu.make_async_copy(
                outb[b], t8_hbm.at[pl.ds(tc * CW * DP, CW * DP)],
                wsem[b]).start()

    def write_wait(b, i):
        @pl.when(jnp.logical_and(i >= 0, cid(i) < FULLC))
        def _():
            pltpu.make_async_copy(
                outb[b], t8_hbm.at[pl.ds(0, CW * DP)], wsem[b]).wait()

    read_start(0, 0)
    read_start(1, 1)

    def outer(i2, carry):
        for b in range(2):
            i = i2 * 2 + b
            read_wait(b, i)
            write_wait(b, i - 2)

            @pl.when(cid(i) < FULLC)
            def _():
                transpose(b)

            write_start(b, i)
            read_start(b, i + 2)
        return carry

    lax.fori_loop(0, AIT // 2, outer, 0)
    write_wait(0, AIT - 2)
    write_wait(1, AIT - 1)

    @pl.when(wid == 0)
    def _():
        pltpu.sync_copy(tail_hbm, outb[0].at[pl.ds(0, 64 * DP)])
        pltpu.sync_copy(outb[0].at[pl.ds(0, 64 * DP)],
                        t8_hbm.at[pl.ds(FULLC * CW * DP, 64 * DP)])


@functools.partial(
    pl.kernel,
    mesh=_mesh,
    compiler_params=pltpu.CompilerParams(
        use_tc_tiling_on_sc=False, needs_layout_passes=False),
    out_type=jax.ShapeDtypeStruct((L * 8 * NW * 8 * BP,), jnp.float32),
    scratch_types=[
        pltpu.VMEM((L // 4, 4 * BP), jnp.int32),
        [pltpu.VMEM((4 * BP, DP), jnp.float32) for _ in range(2)],
        pltpu.VMEM((4 * D * BP,), jnp.float32),
        pltpu.SemaphoreType.DMA,
        [pltpu.SemaphoreType.DMA for _ in range(2)],
        pltpu.SemaphoreType.DMA,
    ],
)
def _embed_sc(x_hbm, table_hbm, out_hbm, idx_v, rows, trans, ssem, gsem, wsem):
    wid = lax.axis_index("s") * NUM_CORES + lax.axis_index("c")

    def stage(l, carry):
        lt, lp = l // 8, l % 8
        pltpu.make_async_copy(
            x_hbm.at[pl.ds(lt * NW * 8 + wid * 8 + lp, 1)],
            idx_v.at[pl.ds(l // 4, 1), pl.ds((l % 4) * BP, BP)],
            ssem).start()
        return carry

    lax.fori_loop(0, L, stage, 0, unroll=8)
    pltpu.make_async_copy(
        x_hbm.at[pl.ds(0, L // 4)], idx_v, ssem).wait()

    iota = lax.broadcasted_iota(jnp.int32, (16,), 0)
    d_base = [(iota + (16 * k)) * BP for k in range(D // 16)]

    def gather_start(b, c):
        pltpu.make_async_copy(
            table_hbm.at[idx_v.at[c]], rows[b], gsem[b]).start()

    def gather_wait(b):
        pltpu.make_async_copy(
            table_hbm.at[idx_v.at[0]], rows[b], gsem[b]).wait()

    def transpose_quarter(b, q):
        src, dst = rows[b], trans

        def body(bp2, carry):
            bp = bp2 * 2
            vals = [src[q * BP + bp + g // 4, pl.ds(16 * (g % 4), 16)]
                    for g in range(8)]
            idxs = [d_base[g % 4] + (bp + g // 4) for g in range(8)]
            for g in range(8):
                plsc.store_scatter(dst, [q * D * BP + idxs[g]], vals[g])
            return carry

        lax.fori_loop(0, BP // 2, body, 0, unroll=4)

    def write_quarter(c, q):
        l = c * 4 + q
        for t in range(8):
            r0 = ((l * 8 + t) * NW + wid) * 8 * BP
            pltpu.make_async_copy(
                trans.at[pl.ds((q * D + t * 8) * BP, 8 * BP)],
                out_hbm.at[pl.ds(r0, 8 * BP)],
                wsem).start()

    def write_wait():
        pltpu.make_async_copy(
            trans, out_hbm.at[pl.ds(0, 4 * D * BP)], wsem).wait()

    NCH = L // 4
    gather_start(0, 0)
    gather_start(1, 1)

    def outer(i, carry):
        for b in range(2):
            c = i * 2 + b
            gather_wait(b)

            @pl.when(c >= 1)
            def _():
                write_wait()

            for q in range(4):
                transpose_quarter(b, q)
                write_quarter(c, q)

            @pl.when(c + 2 < NCH)
            def _():
                gather_start(b, c + 2)
        return carry

    lax.fori_loop(0, NCH // 2, outer, 0)
    write_wait()


def kernel(x, table):
    tt = table.T
    x4 = x.reshape(NW, BP, LT, 8).transpose(2, 0, 3, 1).reshape(LT * NW * 8, BP)
    tail = (table[FULLC * CW:] * SCALE).reshape(64 * DP)
    t8 = _transpose_table(tt, tail)
    o2 = _embed_sc(x4, t8.reshape(VOCAB, DP))
    out = (o2.reshape(L, 8, NW, 8, BP)
              .transpose(2, 4, 0, 1, 3)
              .reshape(B, L, D))
    return out


# speedup vs baseline: 1.6526x; 1.6526x over previous
import functools
import math

import jax
import jax.numpy as jnp
from jax import lax
from jax.experimental import pallas as pl
from jax.experimental.pallas import tpu as pltpu
from jax.experimental.pallas import tpu_sc as plsc

VOCAB = 1000000
D = 64
B = 4096
L = 200
SCALE = math.sqrt(D)
DP = 72

NUM_CORES = 2
NUM_SUBCORES = 16
NW = NUM_CORES * NUM_SUBCORES
BP = 128
LT = L // 8
NBUF = 4

CW = 256
FULLC = VOCAB // CW
AIT = FULLC // NW + 2

_mesh = plsc.VectorSubcoreMesh(core_axis_name="c", subcore_axis_name="s")


@functools.partial(
    pl.kernel,
    mesh=_mesh,
    compiler_params=pltpu.CompilerParams(needs_layout_passes=False),
    out_type=jax.ShapeDtypeStruct((VOCAB * DP,), jnp.float32),
    scratch_types=[
        [pltpu.VMEM((D, CW), jnp.float32) for _ in range(2)],
        [pltpu.VMEM((CW * DP,), jnp.float32) for _ in range(2)],
        [pltpu.SemaphoreType.DMA for _ in range(2)],
        [pltpu.SemaphoreType.DMA for _ in range(2)],
    ],
)
def _transpose_table(tt_hbm, tail_hbm, t8_hbm, inb, outb, rsem, wsem):
    wid = lax.axis_index("s") * NUM_CORES + lax.axis_index("c")

    iota = lax.broadcasted_iota(jnp.int32, (16,), 0)
    mbase = [(iota + (16 * m)) * DP for m in range(CW // 16)]

    def cid(i):
        return i * NW + wid

    def read_start(b, i):
        tc = cid(i)

        @pl.when(tc < FULLC)
        def _():
            pltpu.make_async_copy(
                tt_hbm.at[:, pl.ds(tc * CW, CW)], inb[b], rsem[b]).start()

    def read_wait(b, i):
        @pl.when(cid(i) < FULLC)
        def _():
            pltpu.make_async_copy(
                tt_hbm.at[:, pl.ds(0, CW)], inb[b], rsem[b]).wait()

    def transpose(b):
        src, dst = inb[b], outb[b]

        def dbody(d, carry):
            vals = [src[d, pl.ds(16 * m, 16)] * SCALE
                    for m in range(CW // 16)]
            idxs = [mbase[m] + d for m in range(CW // 16)]
            for m in range(CW // 16):
                plsc.store_scatter(dst, [idxs[m]], vals[m])
            return carry

        lax.fori_loop(0, D, dbody, 0, unroll=2)

    def write_start(b, i):
        tc = cid(i)

        @pl.when(tc < FULLC)
        def _():
            pltpu.make_async_copy(
                outb[b], t8_hbm.at[pl.ds(tc * CW * DP, CW * DP)],
                wsem[b]).start()

    def write_wait(b, i):
        @pl.when(jnp.logical_and(i >= 0, cid(i) < FULLC))
        def _():
            pltpu.make_async_copy(
                outb[b], t8_hbm.at[pl.ds(0, CW * DP)], wsem[b]).wait()

    read_start(0, 0)
    read_start(1, 1)

    def outer(i2, carry):
        for b in range(2):
            i = i2 * 2 + b
            read_wait(b, i)
            write_wait(b, i - 2)

            @pl.when(cid(i) < FULLC)
            def _():
                transpose(b)

            write_start(b, i)
            read_start(b, i + 2)
        return carry

    lax.fori_loop(0, AIT // 2, outer, 0)
    write_wait(0, AIT - 2)
    write_wait(1, AIT - 1)

    @pl.when(wid == 0)
    def _():
        pltpu.sync_copy(tail_hbm, outb[0].at[pl.ds(0, 64 * DP)])
        pltpu.sync_copy(outb[0].at[pl.ds(0, 64 * DP)],
                        t8_hbm.at[pl.ds(FULLC * CW * DP, 64 * DP)])


@functools.partial(
    pl.kernel,
    mesh=_mesh,
    compiler_params=pltpu.CompilerParams(
        use_tc_tiling_on_sc=False, needs_layout_passes=False),
    out_type=jax.ShapeDtypeStruct((L * 8 * NW * 8 * BP,), jnp.float32),
    scratch_types=[
        pltpu.VMEM((L // 4, 4 * BP), jnp.int32),
        [pltpu.VMEM((4 * BP, DP), jnp.float32) for _ in range(2)],
        [pltpu.VMEM((D * BP,), jnp.float32) for _ in range(2)],
        pltpu.SemaphoreType.DMA,
        [pltpu.SemaphoreType.DMA for _ in range(2)],
        [pltpu.SemaphoreType.DMA for _ in range(2)],
    ],
)
def _embed_sc(x_hbm, table_hbm, out_hbm, idx_v, rows, trans, ssem, gsem, wsem):
    wid = lax.axis_index("s") * NUM_CORES + lax.axis_index("c")

    def stage(l, carry):
        lt, lp = l // 8, l % 8
        pltpu.make_async_copy(
            x_hbm.at[pl.ds(lt * NW * 8 + wid * 8 + lp, 1)],
            idx_v.at[pl.ds(l // 4, 1), pl.ds((l % 4) * BP, BP)],
            ssem).start()
        return carry

    lax.fori_loop(0, L, stage, 0, unroll=8)
    pltpu.make_async_copy(
        x_hbm.at[pl.ds(0, L // 4)], idx_v, ssem).wait()

    iota = lax.broadcasted_iota(jnp.int32, (16,), 0)
    d_base = [(iota + (16 * k)) * BP for k in range(D // 16)]

    def gather_start(b, c):
        pltpu.make_async_copy(
            table_hbm.at[idx_v.at[c]], rows[b], gsem[b]).start()

    def gather_wait(b):
        pltpu.make_async_copy(
            table_hbm.at[idx_v.at[0]], rows[b], gsem[b]).wait()

    def transpose_quarter(b, q):
        src, dst = rows[b], trans[q % 2]

        def body(bp2, carry):
            bp = bp2 * 2
            vals = [src[q * BP + bp + g // 4, pl.ds(16 * (g % 4), 16)]
                    for g in range(8)]
            idxs = [d_base[g % 4] + (bp + g // 4) for g in range(8)]
            for g in range(8):
                plsc.store_scatter(dst, [idxs[g]], vals[g])
            return carry

        lax.fori_loop(0, BP // 2, body, 0, unroll=4)

    def write_quarter(c, q):
        l = c * 4 + q
        for t in range(8):
            r0 = ((l * 8 + t) * NW + wid) * 8 * BP
            pltpu.make_async_copy(
                trans[q % 2].at[pl.ds(t * 8 * BP, 8 * BP)],
                out_hbm.at[pl.ds(r0, 8 * BP)],
                wsem[q % 2]).start()

    def write_wait(h):
        pltpu.make_async_copy(
            trans[h], out_hbm.at[pl.ds(0, D * BP)], wsem[h]).wait()

    NCH = L // 4
    gather_start(0, 0)
    gather_start(1, 1)

    def outer(i, carry):
        for b in range(2):
            c = i * 2 + b
            gather_wait(b)
            for q in range(4):
                @pl.when(jnp.logical_or(c >= 1, q >= 2))
                def _():
                    write_wait(q % 2)
                transpose_quarter(b, q)
                write_quarter(c, q)

            @pl.when(c + 2 < NCH)
            def _():
                gather_start(b, c + 2)
        return carry

    lax.fori_loop(0, NCH // 2, outer, 0)
    write_wait(0)
    write_wait(1)


def kernel(x, table):
    tt = table.T
    x4 = x.reshape(NW, BP, LT, 8).transpose(2, 0, 3, 1).reshape(LT * NW * 8, BP)
    tail = jnp.pad(table[FULLC * CW:] * SCALE,
                   ((0, 0), (0, DP - D))).reshape(64 * DP)
    t8 = _transpose_table(tt, tail)
    o2 = _embed_sc(x4, t8.reshape(VOCAB, DP))
    out = (o2.reshape(L, 8, NW, 8, BP)
              .transpose(2, 4, 0, 1, 3)
              .reshape(B, L, D))
    return out


# speedup vs baseline: 3.2769x; 1.9829x over previous
import functools
import math

import jax
import jax.numpy as jnp
from jax import lax
from jax.experimental import pallas as pl
from jax.experimental.pallas import tpu as pltpu
from jax.experimental.pallas import tpu_sc as plsc

VOCAB = 1000000
D = 64
B = 4096
L = 200
SCALE = math.sqrt(D)
DP = 72

NUM_CORES = 2
NUM_SUBCORES = 16
NW = NUM_CORES * NUM_SUBCORES
BP = 128
LT = L // 8
NBUF = 4

CW = 256
FULLC = VOCAB // CW
AIT = FULLC // NW + 2

_mesh = plsc.VectorSubcoreMesh(core_axis_name="c", subcore_axis_name="s")


@functools.partial(
    pl.kernel,
    mesh=_mesh,
    compiler_params=pltpu.CompilerParams(needs_layout_passes=False),
    out_type=jax.ShapeDtypeStruct((VOCAB * DP,), jnp.float32),
    scratch_types=[
        [pltpu.VMEM((D, CW), jnp.float32) for _ in range(2)],
        [pltpu.VMEM((CW * DP,), jnp.float32) for _ in range(2)],
        [pltpu.SemaphoreType.DMA for _ in range(2)],
        [pltpu.SemaphoreType.DMA for _ in range(2)],
    ],
)
def _transpose_table(tt_hbm, tail_hbm, t8_hbm, inb, outb, rsem, wsem):
    wid = lax.axis_index("s") * NUM_CORES + lax.axis_index("c")

    iota = lax.broadcasted_iota(jnp.int32, (16,), 0)
    mbase = [(iota + (16 * m)) * DP for m in range(CW // 16)]

    def cid(i):
        return i * NW + wid

    def read_start(b, i):
        tc = cid(i)

        @pl.when(tc < FULLC)
        def _():
            pltpu.make_async_copy(
                tt_hbm.at[:, pl.ds(tc * CW, CW)], inb[b], rsem[b]).start()

    def read_wait(b, i):
        @pl.when(cid(i) < FULLC)
        def _():
            pltpu.make_async_copy(
                tt_hbm.at[:, pl.ds(0, CW)], inb[b], rsem[b]).wait()

    def transpose(b):
        src, dst = inb[b], outb[b]

        def dbody(d, carry):
            vals = [src[d, pl.ds(16 * m, 16)] * SCALE
                    for m in range(CW // 16)]
            idxs = [mbase[m] + d for m in range(CW // 16)]
            for m in range(CW // 16):
                plsc.store_scatter(dst, [idxs[m]], vals[m])
            return carry

        lax.fori_loop(0, D, dbody, 0, unroll=2)

    def write_start(b, i):
        tc = cid(i)

        @pl.when(tc < FULLC)
        def _():
            pltpu.make_async_copy(
                outb[b], t8_hbm.at[pl.ds(tc * CW * DP, CW * DP)],
                wsem[b]).start()

    def write_wait(b, i):
        @pl.when(jnp.logical_and(i >= 0, cid(i) < FULLC))
        def _():
            pltpu.make_async_copy(
                outb[b], t8_hbm.at[pl.ds(0, CW * DP)], wsem[b]).wait()

    read_start(0, 0)
    read_start(1, 1)

    def outer(i2, carry):
        for b in range(2):
            i = i2 * 2 + b
            read_wait(b, i)
            write_wait(b, i - 2)

            @pl.when(cid(i) < FULLC)
            def _():
                transpose(b)

            write_start(b, i)
            read_start(b, i + 2)
        return carry

    lax.fori_loop(0, AIT // 2, outer, 0)
    write_wait(0, AIT - 2)
    write_wait(1, AIT - 1)

    @pl.when(wid == 0)
    def _():
        pltpu.sync_copy(tail_hbm, outb[0].at[pl.ds(0, 64 * DP)])
        pltpu.sync_copy(outb[0].at[pl.ds(0, 64 * DP)],
                        t8_hbm.at[pl.ds(FULLC * CW * DP, 64 * DP)])


@functools.partial(
    pl.kernel,
    mesh=_mesh,
    compiler_params=pltpu.CompilerParams(
        use_tc_tiling_on_sc=False, needs_layout_passes=False),
    out_type=jax.ShapeDtypeStruct((L * 8 * NW * 8 * BP,), jnp.float32),
    scratch_types=[
        pltpu.VMEM((L // 4, 4 * BP), jnp.int32),
        [pltpu.VMEM((4 * BP, DP), jnp.float32) for _ in range(2)],
        [pltpu.VMEM((D * BP,), jnp.float32) for _ in range(2)],
        pltpu.SemaphoreType.DMA,
        [pltpu.SemaphoreType.DMA for _ in range(2)],
        [pltpu.SemaphoreType.DMA for _ in range(2)],
    ],
)
def _embed_sc(x_hbm, table_hbm, out_hbm, idx_v, rows, trans, ssem, gsem, wsem):
    wid = lax.axis_index("s") * NUM_CORES + lax.axis_index("c")

    def stage(l, carry):
        lt, lp = l // 8, l % 8
        pltpu.make_async_copy(
            x_hbm.at[pl.ds(lt * NW * 8 + wid * 8 + lp, 1)],
            idx_v.at[pl.ds(l // 4, 1), pl.ds((l % 4) * BP, BP)],
            ssem).start()
        return carry

    lax.fori_loop(0, L, stage, 0, unroll=8)
    pltpu.make_async_copy(
        x_hbm.at[pl.ds(0, L // 4)], idx_v, ssem).wait()

    iota = lax.broadcasted_iota(jnp.int32, (16,), 0)
    m_row = [iota + (16 * m) for m in range(BP // 16)]

    def gather_start(b, c):
        pltpu.make_async_copy(
            table_hbm.at[idx_v.at[c]], rows[b], gsem[b]).start()

    def gather_wait(b):
        pltpu.make_async_copy(
            table_hbm.at[idx_v.at[0]], rows[b], gsem[b]).wait()

    def transpose_quarter(b, q):
        src, dst = rows[b], trans[q % 2]
        rws = [m_row[m] + (q * BP) for m in range(BP // 16)]

        def dbody(d, carry):
            col = jnp.full((16,), d, jnp.int32)
            vals = [plsc.load_gather(src, [rws[m], col])
                    for m in range(BP // 16)]
            for m in range(BP // 16):
                dst[pl.ds(d * BP + 16 * m, 16)] = vals[m]
            return carry

        lax.fori_loop(0, D, dbody, 0, unroll=2)

    def write_quarter(c, q):
        l = c * 4 + q
        for t in range(8):
            r0 = ((l * 8 + t) * NW + wid) * 8 * BP
            pltpu.make_async_copy(
                trans[q % 2].at[pl.ds(t * 8 * BP, 8 * BP)],
                out_hbm.at[pl.ds(r0, 8 * BP)],
                wsem[q % 2]).start()

    def write_wait(h):
        pltpu.make_async_copy(
            trans[h], out_hbm.at[pl.ds(0, D * BP)], wsem[h]).wait()

    NCH = L // 4
    gather_start(0, 0)
    gather_start(1, 1)

    def outer(i, carry):
        for b in range(2):
            c = i * 2 + b
            gather_wait(b)
            for q in range(4):
                @pl.when(jnp.logical_or(c >= 1, q >= 2))
                def _():
                    write_wait(q % 2)
                transpose_quarter(b, q)
                write_quarter(c, q)

            @pl.when(c + 2 < NCH)
            def _():
                gather_start(b, c + 2)
        return carry

    lax.fori_loop(0, NCH // 2, outer, 0)
    write_wait(0)
    write_wait(1)


def kernel(x, table):
    tt = table.T
    x4 = x.reshape(NW, BP, LT, 8).transpose(2, 0, 3, 1).reshape(LT * NW * 8, BP)
    tail = jnp.pad(table[FULLC * CW:] * SCALE,
                   ((0, 0), (0, DP - D))).reshape(64 * DP)
    t8 = _transpose_table(tt, tail)
    o2 = _embed_sc(x4, t8.reshape(VOCAB, DP))
    out = (o2.reshape(L, 8, NW, 8, BP)
              .transpose(2, 4, 0, 1, 3)
              .reshape(B, L, D))
    return out
